# baseline (speedup 1.0000x reference)
import functools

import jax
import jax.numpy as jnp
from jax import lax
from jax.experimental import pallas as pl
from jax.experimental.pallas import tpu as pltpu
from jax.experimental.pallas import tpu_sc as plsc

N = 10000
E = 320000
D = 128
DF = 16

NC = 2
NS = 16
NW = NC * NS
EW = E // NW
CHUNK = 40
NCHUNKS = EW // CHUNK
NBUF = 8
REM = NCHUNKS - ((NCHUNKS - 1) // NBUF) * NBUF
RPT = 624
TAIL = N - RPT * NS

_MESH = plsc.VectorSubcoreMesh(core_axis_name="c", subcore_axis_name="s")


def _sc_scatter_mean_partials(dest, edge_attr, zsum, zcnt, ones):
    @functools.partial(
        pl.kernel,
        out_type=(
            jax.ShapeDtypeStruct((NC, N, D), jnp.float32),
            jax.ShapeDtypeStruct((NC * N,), jnp.float32),
        ),
        mesh=_MESH,
        scratch_types=(
            [pltpu.VMEM((CHUNK,), jnp.int32)] * NBUF
            + [pltpu.VMEM((CHUNK, D), jnp.float32)] * NBUF
            + [
                pltpu.VMEM((CHUNK,), jnp.float32),
                pltpu.VMEM((RPT,), jnp.float32),
                pltpu.VMEM_SHARED((N, D), jnp.float32),
                pltpu.VMEM_SHARED((N,), jnp.float32),
            ]
            + [pltpu.SemaphoreType.DMA] * (2 * NBUF)
        ),
    )
    def body(dest_hbm, attr_hbm, zsum_hbm, zcnt_hbm, ones_hbm,
             sum_out, cnt_out, *scratch):
        idx = scratch[:NBUF]
        rows = scratch[NBUF:2 * NBUF]
        ones_v, stage1_v, ssum, scnt = scratch[2 * NBUF:2 * NBUF + 4]
        ld = scratch[2 * NBUF + 4:2 * NBUF + 4 + NBUF]
        st = scratch[2 * NBUF + 4 + NBUF:]
        cid = lax.axis_index("c")
        sid = lax.axis_index("s")
        wid = cid * NS + sid
        r0 = sid * RPT
        t0 = RPT * NS
        slices = [(CHUNK * k, CHUNK) for k in range(RPT // CHUNK)]
        if RPT % CHUNK:
            slices.append(((RPT // CHUNK) * CHUNK, RPT % CHUNK))
        rows_v0 = rows[0]

        pltpu.sync_copy(zsum_hbm, rows_v0)
        pltpu.sync_copy(zcnt_hbm, stage1_v)
        zs = [pltpu.async_copy(rows_v0.at[pl.ds(0, sz), :],
                               ssum.at[pl.ds(r0 + o, sz), :], ld[0])
              for o, sz in slices]
        for z in zs:
            z.wait()
        pltpu.sync_copy(stage1_v, scnt.at[pl.ds(r0, RPT)])

        @pl.when(sid == 0)
        def _():
            pltpu.sync_copy(rows_v0.at[pl.ds(0, TAIL), :],
                            ssum.at[pl.ds(t0, TAIL), :])
            pltpu.sync_copy(stage1_v.at[pl.ds(0, TAIL)],
                            scnt.at[pl.ds(t0, TAIL)])

        pltpu.sync_copy(ones_hbm, ones_v)
        plsc.subcore_barrier()

        base = wid * EW

        def start_load(off, b):
            pltpu.async_copy(dest_hbm.at[pl.ds(off, CHUNK)], idx[b], ld[b])
            pltpu.async_copy(attr_hbm.at[pl.ds(off, CHUNK), :], rows[b], ld[b])

        def wait_load(b):
            pltpu.make_async_copy(dest_hbm.at[pl.ds(base, CHUNK)], idx[b],
                                  ld[b]).wait()
            pltpu.make_async_copy(attr_hbm.at[pl.ds(base, CHUNK), :], rows[b],
                                  ld[b]).wait()

        def start_scat(b):
            pltpu.async_copy(rows[b], ssum.at[idx[b]], st[b], add=True)
            pltpu.async_copy(ones_v, scnt.at[idx[b]], st[b], add=True)

        def wait_scat(b):
            pltpu.make_async_copy(rows[b], ssum.at[idx[b]], st[b]).wait()
            pltpu.make_async_copy(ones_v, scnt.at[idx[b]], st[b]).wait()

        maxoff = base + (NCHUNKS - 1) * CHUNK
        for b in range(NBUF):
            start_load(base + b * CHUNK, b)

        @pl.loop(0, NCHUNKS - REM, step=NBUF)
        def _(j):
            off = base + j * CHUNK
            for b in range(NBUF):
                wait_load(b)
                start_scat(b)
            for b in range(NBUF):
                wait_scat(b)
                start_load(jnp.minimum(off + (NBUF + b) * CHUNK, maxoff), b)

        for b in range(REM):
            wait_load(b)
            start_scat(b)
        for b in range(REM, NBUF):
            wait_load(b)
        for b in range(REM):
            wait_scat(b)

        plsc.subcore_barrier()
        outs = []
        for k, (o, sz) in enumerate(slices):
            b = k % 2
            if k >= 2:
                outs[k - 2].wait()
            pltpu.sync_copy(ssum.at[pl.ds(r0 + o, sz), :],
                            rows[b].at[pl.ds(0, sz), :])
            outs.append(pltpu.async_copy(
                rows[b].at[pl.ds(0, sz), :],
                sum_out.at[cid, pl.ds(r0 + o, sz), :], st[b]))
        outs[-2].wait()
        outs[-1].wait()
        pltpu.sync_copy(scnt.at[pl.ds(r0, RPT)], stage1_v)
        pltpu.sync_copy(stage1_v, cnt_out.at[pl.ds(cid * N + r0, RPT)])

        @pl.when(sid == 0)
        def _():
            pltpu.sync_copy(ssum.at[pl.ds(t0, TAIL), :],
                            rows_v0.at[pl.ds(0, TAIL), :])
            pltpu.sync_copy(scnt.at[pl.ds(t0, TAIL)],
                            stage1_v.at[pl.ds(0, TAIL)])
            pltpu.sync_copy(rows_v0.at[pl.ds(0, TAIL), :],
                            sum_out.at[cid, pl.ds(t0, TAIL), :])
            pltpu.sync_copy(stage1_v.at[pl.ds(0, TAIL)],
                            cnt_out.at[pl.ds(cid * N + t0, TAIL)])

    return body(dest, edge_attr, zsum, zcnt, ones)


def _mlp_block(x_ref, f_ref, s2_ref, c2_ref, w1x_ref, w1a_ref, w1f_ref,
               b1_ref, w2_ref, b2_ref, w3_ref, b3_ref, out_ref):
    s = s2_ref[0] + s2_ref[1]
    c = c2_ref[:, 0:1] + c2_ref[:, 1:2]
    inv = 1.0 / jnp.maximum(c, 1.0)
    h = (jnp.dot(x_ref[...], w1x_ref[...], preferred_element_type=jnp.float32)
         + jnp.dot(s, w1a_ref[...], preferred_element_type=jnp.float32) * inv
         + jnp.dot(f_ref[...], w1f_ref[...], preferred_element_type=jnp.float32)
         + b1_ref[...])
    h = h * jax.nn.sigmoid(h)
    h = jnp.dot(h, w2_ref[...], preferred_element_type=jnp.float32) + b2_ref[...]
    h = h * jax.nn.sigmoid(h)
    out_ref[...] = (jnp.dot(h, w3_ref[...], preferred_element_type=jnp.float32)
                    + b3_ref[...])


def _tc_mlp(x, f, sums2, cnt2, w1x, w1a, w1f, b1, w2, b2, w3, b3):
    B = 1000
    grid = (N // B,)
    return pl.pallas_call(
        _mlp_block,
        grid=grid,
        in_specs=[
            pl.BlockSpec((B, D), lambda i: (i, 0)),
            pl.BlockSpec((B, DF), lambda i: (i, 0)),
            pl.BlockSpec((NC, B, D), lambda i: (0, i, 0)),
            pl.BlockSpec((B, NC), lambda i: (i, 0)),
            pl.BlockSpec((D, D), lambda i: (0, 0)),
            pl.BlockSpec((D, D), lambda i: (0, 0)),
            pl.BlockSpec((DF, D), lambda i: (0, 0)),
            pl.BlockSpec((1, D), lambda i: (0, 0)),
            pl.BlockSpec((D, D), lambda i: (0, 0)),
            pl.BlockSpec((1, D), lambda i: (0, 0)),
            pl.BlockSpec((D, D), lambda i: (0, 0)),
            pl.BlockSpec((1, D), lambda i: (0, 0)),
        ],
        out_specs=pl.BlockSpec((B, D), lambda i: (i, 0)),
        out_shape=jax.ShapeDtypeStruct((N, D), jnp.float32),
    )(x, f, sums2, cnt2, w1x, w1a, w1f, b1, w2, b2, w3, b3)


def kernel(x, edge_index, edge_attr, f, W1, b1, W2, b2, W3, b3):
    dest = edge_index[1]
    zsum = jnp.zeros((CHUNK, D), jnp.float32)
    zcnt = jnp.zeros((RPT,), jnp.float32)
    ones = jnp.ones((CHUNK,), jnp.float32)
    sums2, cnt_flat = _sc_scatter_mean_partials(dest, edge_attr, zsum, zcnt,
                                                ones)
    cnt2 = cnt_flat.reshape(NC, N).T

    w1t = W1.T
    w1x = w1t[:D]
    w1a = w1t[D:2 * D]
    w1f = w1t[2 * D:]
    return _tc_mlp(x, f, sums2, cnt2, w1x, w1a, w1f,
                   b1.reshape(1, D), W2.T, b2.reshape(1, D),
                   W3.T, b3.reshape(1, D))


# speedup vs baseline: 1.1065x; 1.1065x over previous
import functools

import jax
import jax.numpy as jnp
from jax import lax
from jax.experimental import pallas as pl
from jax.experimental.pallas import tpu as pltpu
from jax.experimental.pallas import tpu_sc as plsc

N = 10000
E = 320000
D = 128
DF = 16

NC = 2
NS = 16
NW = NC * NS
EW = E // NW
CHUNK = 40
NCHUNKS = EW // CHUNK
NBUF = 8
REM = NCHUNKS - ((NCHUNKS - 1) // NBUF) * NBUF
RPT = 624
TAIL = N - RPT * NS

_MESH = plsc.VectorSubcoreMesh(core_axis_name="c", subcore_axis_name="s")


def _sc_scatter_mean_partials(dest, edge_attr, zsum, zcnt, ones):
    @functools.partial(
        pl.kernel,
        out_type=(
            jax.ShapeDtypeStruct((NC, N, D), jnp.float32),
            jax.ShapeDtypeStruct((NC * N,), jnp.float32),
        ),
        mesh=_MESH,
        scratch_types=(
            [pltpu.VMEM((CHUNK,), jnp.int32)] * NBUF
            + [pltpu.VMEM((CHUNK, D), jnp.float32)] * NBUF
            + [
                pltpu.VMEM((CHUNK,), jnp.float32),
                pltpu.VMEM((RPT,), jnp.float32),
                pltpu.VMEM_SHARED((N, D), jnp.float32),
                pltpu.VMEM_SHARED((N,), jnp.float32),
            ]
            + [pltpu.SemaphoreType.DMA] * (2 * NBUF)
        ),
    )
    def body(dest_hbm, attr_hbm, zsum_hbm, zcnt_hbm, ones_hbm,
             sum_out, cnt_out, *scratch):
        idx = scratch[:NBUF]
        rows = scratch[NBUF:2 * NBUF]
        ones_v, stage1_v, ssum, scnt = scratch[2 * NBUF:2 * NBUF + 4]
        ld = scratch[2 * NBUF + 4:2 * NBUF + 4 + NBUF]
        st = scratch[2 * NBUF + 4 + NBUF:]
        cid = lax.axis_index("c")
        sid = lax.axis_index("s")
        wid = cid * NS + sid
        r0 = sid * RPT
        t0 = RPT * NS
        slices = [(CHUNK * k, CHUNK) for k in range(RPT // CHUNK)]
        if RPT % CHUNK:
            slices.append(((RPT // CHUNK) * CHUNK, RPT % CHUNK))
        rows_v0 = rows[0]

        pltpu.sync_copy(zsum_hbm, rows_v0)
        pltpu.sync_copy(zcnt_hbm, stage1_v)
        zs = [pltpu.async_copy(rows_v0.at[pl.ds(0, sz), :],
                               ssum.at[pl.ds(r0 + o, sz), :], ld[0])
              for o, sz in slices]
        for z in zs:
            z.wait()
        pltpu.sync_copy(stage1_v, scnt.at[pl.ds(r0, RPT)])

        @pl.when(sid == 0)
        def _():
            pltpu.sync_copy(rows_v0.at[pl.ds(0, TAIL), :],
                            ssum.at[pl.ds(t0, TAIL), :])
            pltpu.sync_copy(stage1_v.at[pl.ds(0, TAIL)],
                            scnt.at[pl.ds(t0, TAIL)])

        pltpu.sync_copy(ones_hbm, ones_v)
        plsc.subcore_barrier()

        base = wid * EW

        def start_load(off, b):
            pltpu.async_copy(dest_hbm.at[pl.ds(off, CHUNK)], idx[b], ld[b])
            pltpu.async_copy(attr_hbm.at[pl.ds(off, CHUNK), :], rows[b], ld[b])

        def wait_load(b):
            pltpu.make_async_copy(dest_hbm.at[pl.ds(base, CHUNK)], idx[b],
                                  ld[b]).wait()
            pltpu.make_async_copy(attr_hbm.at[pl.ds(base, CHUNK), :], rows[b],
                                  ld[b]).wait()

        def start_scat(b):
            pltpu.async_copy(rows[b], ssum.at[idx[b]], st[b], add=True)
            pltpu.async_copy(ones_v, scnt.at[idx[b]], st[b], add=True)

        def wait_scat(b):
            pltpu.make_async_copy(rows[b], ssum.at[idx[b]], st[b]).wait()
            pltpu.make_async_copy(ones_v, scnt.at[idx[b]], st[b]).wait()

        maxoff = base + (NCHUNKS - 1) * CHUNK
        for b in range(NBUF):
            start_load(base + b * CHUNK, b)

        @pl.loop(0, NCHUNKS - REM, step=NBUF)
        def _(j):
            off = base + j * CHUNK
            for b in range(NBUF):
                wait_load(b)
                start_scat(b)
            for b in range(NBUF):
                wait_scat(b)
                start_load(jnp.minimum(off + (NBUF + b) * CHUNK, maxoff), b)

        for b in range(REM):
            wait_load(b)
            start_scat(b)
        for b in range(REM, NBUF):
            wait_load(b)
        for b in range(REM):
            wait_scat(b)

        plsc.subcore_barrier()
        outs = []
        for k, (o, sz) in enumerate(slices):
            b = k % 2
            if k >= 2:
                outs[k - 2].wait()
            pltpu.sync_copy(ssum.at[pl.ds(r0 + o, sz), :],
                            rows[b].at[pl.ds(0, sz), :])
            outs.append(pltpu.async_copy(
                rows[b].at[pl.ds(0, sz), :],
                sum_out.at[cid, pl.ds(r0 + o, sz), :], st[b]))
        outs[-2].wait()
        outs[-1].wait()
        pltpu.sync_copy(scnt.at[pl.ds(r0, RPT)], stage1_v)
        pltpu.sync_copy(stage1_v, cnt_out.at[pl.ds(cid * N + r0, RPT)])

        @pl.when(sid == 0)
        def _():
            pltpu.sync_copy(ssum.at[pl.ds(t0, TAIL), :],
                            rows_v0.at[pl.ds(0, TAIL), :])
            pltpu.sync_copy(scnt.at[pl.ds(t0, TAIL)],
                            stage1_v.at[pl.ds(0, TAIL)])
            pltpu.sync_copy(rows_v0.at[pl.ds(0, TAIL), :],
                            sum_out.at[cid, pl.ds(t0, TAIL), :])
            pltpu.sync_copy(stage1_v.at[pl.ds(0, TAIL)],
                            cnt_out.at[pl.ds(cid * N + t0, TAIL)])

    return body(dest, edge_attr, zsum, zcnt, ones)


def _row1_block(ei_ref, out_ref):
    out_ref[...] = ei_ref[1]


def _extract_dest(edge_index):
    return pl.pallas_call(
        _row1_block,
        out_shape=jax.ShapeDtypeStruct((E,), jnp.int32),
    )(edge_index)


def _mlp_block(x_ref, f_ref, s2_ref, c2_ref, w1x_ref, w1a_ref, w1f_ref,
               b1_ref, w2_ref, b2_ref, w3_ref, b3_ref, out_ref):
    s = s2_ref[0] + s2_ref[1]
    c = c2_ref[:, 0:1] + c2_ref[:, 1:2]
    inv = 1.0 / jnp.maximum(c, 1.0)
    h = (jnp.dot(x_ref[...], w1x_ref[...], preferred_element_type=jnp.float32)
         + jnp.dot(s, w1a_ref[...], preferred_element_type=jnp.float32) * inv
         + jnp.dot(f_ref[...], w1f_ref[...], preferred_element_type=jnp.float32)
         + b1_ref[...])
    h = h * jax.nn.sigmoid(h)
    h = jnp.dot(h, w2_ref[...], preferred_element_type=jnp.float32) + b2_ref[...]
    h = h * jax.nn.sigmoid(h)
    out_ref[...] = (jnp.dot(h, w3_ref[...], preferred_element_type=jnp.float32)
                    + b3_ref[...])


def _tc_mlp(x, f, sums2, cnt2, w1x, w1a, w1f, b1, w2, b2, w3, b3):
    B = 2000
    grid = (N // B,)
    return pl.pallas_call(
        _mlp_block,
        grid=grid,
        in_specs=[
            pl.BlockSpec((B, D), lambda i: (i, 0)),
            pl.BlockSpec((B, DF), lambda i: (i, 0)),
            pl.BlockSpec((NC, B, D), lambda i: (0, i, 0)),
            pl.BlockSpec((B, NC), lambda i: (i, 0)),
            pl.BlockSpec((D, D), lambda i: (0, 0)),
            pl.BlockSpec((D, D), lambda i: (0, 0)),
            pl.BlockSpec((DF, D), lambda i: (0, 0)),
            pl.BlockSpec((1, D), lambda i: (0, 0)),
            pl.BlockSpec((D, D), lambda i: (0, 0)),
            pl.BlockSpec((1, D), lambda i: (0, 0)),
            pl.BlockSpec((D, D), lambda i: (0, 0)),
            pl.BlockSpec((1, D), lambda i: (0, 0)),
        ],
        out_specs=pl.BlockSpec((B, D), lambda i: (i, 0)),
        out_shape=jax.ShapeDtypeStruct((N, D), jnp.float32),
    )(x, f, sums2, cnt2, w1x, w1a, w1f, b1, w2, b2, w3, b3)


def kernel(x, edge_index, edge_attr, f, W1, b1, W2, b2, W3, b3):
    dest = _extract_dest(edge_index)
    zsum = jnp.zeros((CHUNK, D), jnp.float32)
    zcnt = jnp.zeros((RPT,), jnp.float32)
    ones = jnp.ones((CHUNK,), jnp.float32)
    sums2, cnt_flat = _sc_scatter_mean_partials(dest, edge_attr, zsum, zcnt,
                                                ones)
    cnt2 = cnt_flat.reshape(NC, N).T

    w1t = W1.T
    w1x = w1t[:D]
    w1a = w1t[D:2 * D]
    w1f = w1t[2 * D:]
    return _tc_mlp(x, f, sums2, cnt2, w1x, w1a, w1f,
                   b1.reshape(1, D), W2.T, b2.reshape(1, D),
                   W3.T, b3.reshape(1, D))
